# Initial kernel scaffold; baseline (speedup 1.0000x reference)
import functools
import math

import jax
import jax.numpy as jnp
from jax import lax
from jax.experimental import pallas as pl
from jax.experimental.pallas import tpu as pltpu
from jax.experimental.pallas import tpu_sc as plsc

H = 64
TD = 32
N = 50000
G = 10
PER = N // G
HALF = PER // 2
EG = 800000
EM = 400000
EU = 2 * EM

EP = 802816
IDXR = EP // 128
NW = 32
PW = EP // NW
PT = EP // 16
CH = 512
R = N // 2
DUMMY = R
ACC_ROWS = 25600
EB = 8192
HB = 8000

_mesh = plsc.VectorSubcoreMesh(core_axis_name="c", subcore_axis_name="s")
_f32 = jnp.float32


def _sc_gather(table, idx2d):
    @functools.partial(
        pl.kernel, mesh=_mesh,
        out_type=jax.ShapeDtypeStruct((EP, H), _f32),
        scratch_types=[pltpu.VMEM((4, 128), jnp.int32),
                       pltpu.VMEM((CH, H), _f32),
                       pltpu.SemaphoreType.DMA])
    def k(table_hbm, idx_hbm, out_hbm, iv, rows, sem):
        c = lax.axis_index("c")
        s = lax.axis_index("s")
        w = c * 16 + s

        def body(ch, carry):
            ir0 = w * (PW // 128) + ch * 4
            r0 = w * PW + ch * CH
            pltpu.sync_copy(idx_hbm.at[pl.ds(ir0, 4)], iv)
            cps = [pltpu.async_copy(table_hbm.at[iv.at[j]],
                                    rows.at[pl.ds(j * 128, 128)], sem)
                   for j in range(4)]
            for cp in cps:
                cp.wait()
            pltpu.sync_copy(rows, out_hbm.at[pl.ds(r0, CH)])
            return carry

        lax.fori_loop(0, PW // CH, body, 0)

    return k(table, idx2d)


def _sc_scatter_body(c, s, iv, rows, accum, zeros_hbm, out_hbm, chunk_fn):
    for m in range(4):
        kz = s + 16 * m

        @pl.when(kz < ACC_ROWS // CH)
        def _():
            pltpu.sync_copy(zeros_hbm, accum.at[pl.ds(kz * CH, CH)])

    plsc.subcore_barrier()

    def body(ch, carry):
        chunk_fn(ch)
        return carry

    lax.fori_loop(0, PT // CH, body, 0)
    plsc.subcore_barrier()
    for m in range(4):
        kz = s + 16 * m

        @pl.when(kz < R // 500)
        def _():
            pltpu.sync_copy(accum.at[pl.ds(kz * 500, 500)],
                            out_hbm.at[pl.ds(c * R + kz * 500, 500)])


def _sc_scatter_add(msg, dstsc, zeros):
    @functools.partial(
        pl.kernel, mesh=_mesh,
        out_type=jax.ShapeDtypeStruct((N, H), _f32),
        scratch_types=[pltpu.VMEM((4, 128), jnp.int32),
                       pltpu.VMEM((CH, H), _f32),
                       pltpu.VMEM_SHARED((ACC_ROWS, H), _f32),
                       pltpu.SemaphoreType.DMA])
    def k(msg_hbm, idx_hbm, zeros_hbm, out_hbm, iv, rows, accum, sem):
        del sem
        c = lax.axis_index("c")
        s = lax.axis_index("s")

        def chunk(ch):
            ir0 = s * (PT // 128) + ch * 4
            r0 = s * PT + ch * CH
            pltpu.sync_copy(msg_hbm.at[pl.ds(r0, CH)], rows)
            pltpu.sync_copy(idx_hbm.at[c, pl.ds(ir0, 4)], iv)
            for j in range(4):
                pltpu.sync_copy(rows.at[pl.ds(j * 128, 128)],
                                accum.at[iv.at[j]], add=True)

        _sc_scatter_body(c, s, iv, rows, accum, zeros_hbm, out_hbm, chunk)

    return k(msg, dstsc, zeros)


def _sc_gin_agg(x, src2d, dstsc, zeros):
    @functools.partial(
        pl.kernel, mesh=_mesh,
        out_type=jax.ShapeDtypeStruct((N, H), _f32),
        scratch_types=[pltpu.VMEM((4, 128), jnp.int32),
                       pltpu.VMEM((CH, H), _f32),
                       pltpu.VMEM_SHARED((ACC_ROWS, H), _f32),
                       pltpu.SemaphoreType.DMA])
    def k(x_hbm, src_hbm, idx_hbm, zeros_hbm, out_hbm, iv, rows, accum, sem):
        c = lax.axis_index("c")
        s = lax.axis_index("s")

        def chunk(ch):
            ir0 = s * (PT // 128) + ch * 4
            pltpu.sync_copy(src_hbm.at[pl.ds(ir0, 4)], iv)
            cps = [pltpu.async_copy(x_hbm.at[iv.at[j]],
                                    rows.at[pl.ds(j * 128, 128)], sem)
                   for j in range(4)]
            for cp in cps:
                cp.wait()
            pltpu.sync_copy(idx_hbm.at[c, pl.ds(ir0, 4)], iv)
            for j in range(4):
                pltpu.sync_copy(rows.at[pl.ds(j * 128, 128)],
                                accum.at[iv.at[j]], add=True)

        _sc_scatter_body(c, s, iv, rows, accum, zeros_hbm, out_hbm, chunk)

    return k(x, src2d, dstsc, zeros)


def _full(shape):
    return pl.BlockSpec(shape, lambda *_: tuple(0 for _ in shape))


def _time_body(t_ref, w1, b1, w2, b2, T0, bT0, T1, bT1, te0, te1):
    tcol = t_ref[...]
    j = lax.broadcasted_iota(_f32, (1, TD), 1)
    freqs = jnp.exp(-math.log(10000.0) * j / float(TD))
    a = tcol * freqs
    emb = jnp.concatenate([jnp.cos(a), jnp.sin(a)], axis=1)
    h = jnp.maximum(emb @ w1[...] + b1[...], 0.0)
    tm = h @ w2[...] + b2[...]
    te0[...] = tm @ T0[...] + bT0[...]
    te1[...] = tm @ T1[...] + bT1[...]


def _time_emb(tpad, tp, l0, l1):
    out = pl.pallas_call(
        _time_body,
        in_specs=[_full((16, 1)), _full((H, TD)), _full((1, TD)),
                  _full((TD, TD)), _full((1, TD)),
                  _full((TD, H)), _full((1, H)), _full((TD, H)), _full((1, H))],
        out_specs=[_full((16, H)), _full((16, H))],
        out_shape=[jax.ShapeDtypeStruct((16, H), _f32)] * 2,
    )
    return out(tpad, tp["W1"], tp["b1"][None, :], tp["W2"], tp["b2"][None, :],
               l0["T"], l0["bT"][None, :], l1["T"], l1["bT"][None, :])


def _embed_body(au_ref, W, b, o_ref):
    ji = lax.broadcasted_iota(jnp.int32, (1, H), 1)
    jf = jnp.floor(ji.astype(_f32) / 2.0)
    inv_dim_t = jnp.exp(-(math.log(10000.0) * 2.0 / float(H)) * jf)
    off = jnp.where(ji % 2 == 1, math.pi / 2.0, 0.0).astype(_f32)
    pos = au_ref[...] * inv_dim_t
    o_ref[...] = jnp.sin(pos + off) @ W[...] + b[...]


def _edge_embed(au, ep):
    out = pl.pallas_call(
        _embed_body,
        grid=(EP // EB,),
        in_specs=[pl.BlockSpec((EB, 1), lambda i: (i, 0)),
                  _full((H, H)), _full((1, H))],
        out_specs=pl.BlockSpec((EB, H), lambda i: (i, 0)),
        out_shape=jax.ShapeDtypeStruct((EP, H), _f32),
    )
    return out(au, ep["W"], ep["b"][None, :])


def _node_body(x, agg, te, W1, b1, W2, b2, eps, gw, gb, gms,
               A, bA, B, bB, V, bV, U, bU,
               f_o, xa_o, xb_o, xv_o, xu_o):
    h = (1.0 + eps[0, 0]) * x[...] + agg[...]
    h = jnp.maximum(h @ W1[...] + b1[...], 0.0)
    h = h @ W2[...] + b2[...]
    mean = jnp.mean(h, axis=0, keepdims=True)
    sub = h - mean * gms[...]
    var = jnp.mean(sub * sub, axis=0, keepdims=True)
    f = jnp.maximum(gw[...] * sub * jax.lax.rsqrt(var + 1e-5) + gb[...], 0.0)
    f_o[...] = f
    xa_o[...] = f @ A[...] + bA[...] + te[...]
    xb_o[...] = f @ B[...] + bB[...]
    xv_o[...] = f @ V[...] + bV[...]
    xu_o[...] = f @ U[...] + bU[...]


def _node_dense(x, agg, te, lp):
    g = lp["gin"]
    n = lp["gn"]
    a = lp["agnn"]
    blk = pl.BlockSpec((HALF, H), lambda i: (i, 0))
    out = pl.pallas_call(
        _node_body,
        grid=(N // HALF,),
        in_specs=[blk, blk, pl.BlockSpec((1, H), lambda i: (i // 2, 0)),
                  _full((H, H)), _full((1, H)), _full((H, H)), _full((1, H)),
                  _full((1, 1)), _full((1, H)), _full((1, H)), _full((1, H)),
                  _full((H, H)), _full((1, H)), _full((H, H)), _full((1, H)),
                  _full((H, H)), _full((1, H)), _full((H, H)), _full((1, H))],
        out_specs=[blk] * 5,
        out_shape=[jax.ShapeDtypeStruct((N, H), _f32)] * 5,
    )
    return out(x, agg, te,
               g["W1"], g["b1"][None, :], g["W2"], g["b2"][None, :],
               g["eps"].reshape(1, 1),
               n["weight"][None, :], n["bias"][None, :], n["mean_scale"][None, :],
               a["A"], a["bA"][None, :], a["B"], a["bB"][None, :],
               a["V"], a["bV"][None, :], a["U"], a["bU"][None, :])


def _edge0_body(e, ga, gb, gv, C, bC, e1_o, msg_o):
    en = ga[...] + gb[...] + e[...] @ C[...] + bC[...]
    gate = jax.nn.sigmoid(en)
    msg_o[...] = gate * gv[...]
    e1_o[...] = e[...] + jnp.maximum(en, 0.0)


def _edge_dense0(e, ga, gb, gv, a):
    blk = pl.BlockSpec((EB, H), lambda i: (i, 0))
    out = pl.pallas_call(
        _edge0_body,
        grid=(EP // EB,),
        in_specs=[blk, blk, blk, blk, _full((H, H)), _full((1, H))],
        out_specs=[blk, blk],
        out_shape=[jax.ShapeDtypeStruct((EP, H), _f32)] * 2,
    )
    return out(e, ga, gb, gv, a["C"], a["bC"][None, :])


def _edge1_body(e, ga, gb, C, bC, e1_o):
    en = ga[...] + gb[...] + e[...] @ C[...] + bC[...]
    e1_o[...] = e[...] + jnp.maximum(en, 0.0)


def _edge_dense1(e, ga, gb, a):
    blk = pl.BlockSpec((EB, H), lambda i: (i, 0))
    out = pl.pallas_call(
        _edge1_body,
        grid=(EP // EB,),
        in_specs=[blk, blk, blk, _full((H, H)), _full((1, H))],
        out_specs=blk,
        out_shape=jax.ShapeDtypeStruct((EP, H), _f32),
    )
    return out(e, ga, gb, a["C"], a["bC"][None, :])


def _xupd_body(f, xu, agg2, o):
    o[...] = f[...] + jnp.maximum(xu[...] + agg2[...], 0.0)


def _xupd(f, xu, agg2):
    blk = pl.BlockSpec((HALF, H), lambda i: (i, 0))
    out = pl.pallas_call(
        _xupd_body,
        grid=(N // HALF,),
        in_specs=[blk, blk, blk],
        out_specs=blk,
        out_shape=jax.ShapeDtypeStruct((N, H), _f32),
    )
    return out(f, xu, agg2)


def _head_body(ea, eb, W1, b1, W2, b2, W3, b3, o):
    def mlp(z):
        z = jnp.maximum(z @ W1[...] + b1[...], 0.0)
        z = jnp.maximum(z @ W2[...] + b2[...], 0.0)
        return z @ W3[...] + b3[...]

    o[...] = 0.5 * (mlp(ea[...]) + mlp(eb[...]))


def _head(e2, mp):
    out = pl.pallas_call(
        _head_body,
        grid=(EM // HB,),
        in_specs=[pl.BlockSpec((HB, H), lambda i: (i, 0)),
                  pl.BlockSpec((HB, H), lambda i: (i + EM // HB, 0)),
                  _full((H, 2 * H)), _full((1, 2 * H)),
                  _full((2 * H, H)), _full((1, H)),
                  _full((H, 1)), _full((1, 1))],
        out_specs=pl.BlockSpec((HB, 1), lambda i: (i, 0)),
        out_shape=jax.ShapeDtypeStruct((EM, 1), _f32),
    )
    return out(e2, e2, mp["W1"], mp["b1"][None, :], mp["W2"], mp["b2"][None, :],
               mp["W3"], mp["b3"].reshape(1, 1))


def kernel(x, edge_index, batch, x_indicator, edge_index_mapping,
           noise_mapping_attr, t, params):
    del batch, x_indicator
    padE = EP - EU
    zpad = jnp.zeros((padE,), jnp.int32)
    npad = jnp.full((padE,), N, jnp.int32)

    src_g = jnp.concatenate([edge_index[0], zpad]).reshape(IDXR, 128)
    dst_g = jnp.concatenate([edge_index[1], npad])
    src_u = jnp.concatenate(
        [edge_index_mapping[0], edge_index_mapping[1], zpad]).reshape(IDXR, 128)
    dst_u_raw = jnp.concatenate([edge_index_mapping[1], edge_index_mapping[0]])
    dst_u = jnp.concatenate([dst_u_raw, zpad]).reshape(IDXR, 128)
    dst_u_pad = jnp.concatenate([dst_u_raw, npad])

    def route(d):
        l0 = jnp.where(d < R, d, DUMMY)
        l1 = jnp.where((d >= R) & (d < N), d - R, DUMMY)
        return jnp.stack([l0, l1]).reshape(2, IDXR, 128)

    dstg_sc = route(dst_g)
    dstu_sc = route(dst_u_pad)

    au = jnp.concatenate([noise_mapping_attr, noise_mapping_attr,
                          jnp.zeros((padE,), _f32)]).reshape(EP, 1)
    tpad = jnp.pad(t, (0, 16 - G)).reshape(16, 1)
    zeros = jnp.zeros((CH, H), _f32)

    lp0, lp1 = params["layers"]
    te0, te1 = _time_emb(tpad, params["time"], lp0["agnn"], lp1["agnn"])

    e = _edge_embed(au, params["edge_embed"])

    agg = _sc_gin_agg(x, src_g, dstg_sc, zeros)
    f0, xa0, xb0, xv0, xu0 = _node_dense(x, agg, te0, lp0)
    ga = _sc_gather(xa0, src_u)
    gb = _sc_gather(xb0, dst_u)
    gv = _sc_gather(xv0, src_u)
    e, msg = _edge_dense0(e, ga, gb, gv, lp0["agnn"])
    agg2 = _sc_scatter_add(msg, dstu_sc, zeros)
    x1 = _xupd(f0, xu0, agg2)

    agg_b = _sc_gin_agg(x1, src_g, dstg_sc, zeros)
    _, xa1, xb1, _, _ = _node_dense(x1, agg_b, te1, lp1)
    ga1 = _sc_gather(xa1, src_u)
    gb1 = _sc_gather(xb1, dst_u)
    e = _edge_dense1(e, ga1, gb1, lp1["agnn"])

    return _head(e, params["map"])


# speedup vs baseline: 3.9128x; 3.9128x over previous
import functools
import math

import jax
import jax.numpy as jnp
from jax import lax
from jax.experimental import pallas as pl
from jax.experimental.pallas import tpu as pltpu
from jax.experimental.pallas import tpu_sc as plsc

H = 64
TD = 32
N = 50000
G = 10
PER = N // G
HALF = PER // 2
EG = 800000
EM = 400000
EU = 2 * EM

EP = 819200
IDXR = EP // 128
NW = 32
PW = EP // NW
PT = EP // 16
CHS = 128
R = N // 2
DUMMY = R
ACC_ROWS = 25088
EB = 8192
HB = 8000

_mesh = plsc.VectorSubcoreMesh(core_axis_name="c", subcore_axis_name="s")
_f32 = jnp.float32
_cparams = pltpu.CompilerParams(use_tc_tiling_on_sc=False)


def _sc_gather(table, idx2d, width):
    chw = 32768 // width
    ir = chw // 128
    nch = PW // chw

    @functools.partial(
        pl.kernel, mesh=_mesh, compiler_params=_cparams,
        out_type=jax.ShapeDtypeStruct((EP, width), _f32),
        scratch_types=[pltpu.VMEM((ir, 128), jnp.int32),
                       pltpu.VMEM((ir, 128), jnp.int32),
                       pltpu.VMEM((chw, width), _f32),
                       pltpu.VMEM((chw, width), _f32),
                       pltpu.SemaphoreType.DMA, pltpu.SemaphoreType.DMA,
                       pltpu.SemaphoreType.DMA, pltpu.SemaphoreType.DMA])
    def k(table_hbm, idx_hbm, out_hbm, iva, ivb, ra, rb, ga, gb, sa, sb):
        c = lax.axis_index("c")
        s = lax.axis_index("s")
        w = c * 16 + s
        ib0 = w * (PW // 128)
        rb0 = w * PW

        def load_issue(kk, iv, rows, gsem):
            pltpu.sync_copy(idx_hbm.at[pl.ds(ib0 + kk * ir, ir)], iv)
            for j in range(ir):
                pltpu.async_copy(table_hbm.at[iv.at[j]],
                                 rows.at[pl.ds(j * 128, 128)], gsem)

        load_issue(0, iva, ra, ga)
        load_issue(1, ivb, rb, gb)

        def body(g, carry):
            for kk, iv, rows, gsem, ssem in ((2 * g, iva, ra, ga, sa),
                                             (2 * g + 1, ivb, rb, gb, sb)):
                for j in range(ir):
                    pltpu.make_async_copy(
                        table_hbm.at[iv.at[j]],
                        rows.at[pl.ds(j * 128, 128)], gsem).wait()
                out_slab = out_hbm.at[pl.ds(rb0 + kk * chw, chw)]
                pltpu.async_copy(rows, out_slab, ssem)

                @pl.when(kk + 2 < nch)
                def _():
                    pltpu.make_async_copy(rows, out_slab, ssem).wait()
                    load_issue(kk + 2, iv, rows, gsem)
            return carry

        lax.fori_loop(0, nch // 2, body, 0)
        pltpu.make_async_copy(
            ra, out_hbm.at[pl.ds(rb0 + (nch - 2) * chw, chw)], sa).wait()
        pltpu.make_async_copy(
            rb, out_hbm.at[pl.ds(rb0 + (nch - 1) * chw, chw)], sb).wait()

    return k(table, idx2d)


def _acc_prologue(s, accum, zeros_hbm):
    for m in range(4):
        kz = s + 16 * m

        @pl.when(kz < ACC_ROWS // 512)
        def _():
            pltpu.sync_copy(zeros_hbm, accum.at[pl.ds(kz * 512, 512)])

    plsc.subcore_barrier()


def _acc_epilogue(c, s, accum, out_hbm):
    plsc.subcore_barrier()
    for m in range(2):
        kz = s + 16 * m

        @pl.when(kz < R // 1000)
        def _():
            pltpu.sync_copy(accum.at[pl.ds(kz * 1000, 1000)],
                            out_hbm.at[pl.ds(c * R + kz * 1000, 1000)])


def _sc_scatter_add(msg, dstsc, zeros):
    @functools.partial(
        pl.kernel, mesh=_mesh, compiler_params=_cparams,
        out_type=jax.ShapeDtypeStruct((N, H), _f32),
        scratch_types=[pltpu.VMEM((1, 128), jnp.int32),
                       pltpu.VMEM((1, 128), jnp.int32),
                       pltpu.VMEM((CHS, H), _f32),
                       pltpu.VMEM((CHS, H), _f32),
                       pltpu.VMEM_SHARED((ACC_ROWS, H), _f32),
                       pltpu.SemaphoreType.DMA, pltpu.SemaphoreType.DMA])
    def k(msg_hbm, idx_hbm, zeros_hbm, out_hbm,
          iva, ivb, ra, rb, accum, la, lb):
        c = lax.axis_index("c")
        s = lax.axis_index("s")
        ib0 = s * (PT // 128)
        rb0 = s * PT
        nch = PT // CHS
        _acc_prologue(s, accum, zeros_hbm)

        def issue(kk, rows, lsem):
            pltpu.async_copy(msg_hbm.at[pl.ds(rb0 + kk * CHS, CHS)], rows,
                             lsem)

        issue(0, ra, la)
        issue(1, rb, lb)

        def body(g, carry):
            for kk, iv, rows, lsem in ((2 * g, iva, ra, la),
                                       (2 * g + 1, ivb, rb, lb)):
                pltpu.make_async_copy(
                    msg_hbm.at[pl.ds(rb0 + kk * CHS, CHS)], rows, lsem).wait()
                pltpu.sync_copy(idx_hbm.at[c, pl.ds(ib0 + kk, 1)], iv)
                pltpu.sync_copy(rows, accum.at[iv.at[0]], add=True)

                @pl.when(kk + 2 < nch)
                def _():
                    issue(kk + 2, rows, lsem)
            return carry

        lax.fori_loop(0, nch // 2, body, 0)
        _acc_epilogue(c, s, accum, out_hbm)

    return k(msg, dstsc, zeros)


def _sc_gin_agg(x, src2d, dstsc, zeros):
    @functools.partial(
        pl.kernel, mesh=_mesh, compiler_params=_cparams,
        out_type=jax.ShapeDtypeStruct((N, H), _f32),
        scratch_types=[pltpu.VMEM((1, 128), jnp.int32),
                       pltpu.VMEM((1, 128), jnp.int32),
                       pltpu.VMEM((1, 128), jnp.int32),
                       pltpu.VMEM((1, 128), jnp.int32),
                       pltpu.VMEM((CHS, H), _f32),
                       pltpu.VMEM((CHS, H), _f32),
                       pltpu.VMEM_SHARED((ACC_ROWS, H), _f32),
                       pltpu.SemaphoreType.DMA, pltpu.SemaphoreType.DMA])
    def k(x_hbm, src_hbm, idx_hbm, zeros_hbm, out_hbm,
          isa, isb, iva, ivb, ra, rb, accum, ga, gb):
        c = lax.axis_index("c")
        s = lax.axis_index("s")
        ib0 = s * (PT // 128)
        nch = PT // CHS
        _acc_prologue(s, accum, zeros_hbm)

        def issue(kk, isv, rows, gsem):
            pltpu.sync_copy(src_hbm.at[pl.ds(ib0 + kk, 1)], isv)
            pltpu.async_copy(x_hbm.at[isv.at[0]], rows, gsem)

        issue(0, isa, ra, ga)
        issue(1, isb, rb, gb)

        def body(g, carry):
            for kk, isv, iv, rows, gsem in ((2 * g, isa, iva, ra, ga),
                                            (2 * g + 1, isb, ivb, rb, gb)):
                pltpu.make_async_copy(x_hbm.at[isv.at[0]], rows, gsem).wait()
                pltpu.sync_copy(idx_hbm.at[c, pl.ds(ib0 + kk, 1)], iv)
                pltpu.sync_copy(rows, accum.at[iv.at[0]], add=True)

                @pl.when(kk + 2 < nch)
                def _():
                    issue(kk + 2, isv, rows, gsem)
            return carry

        lax.fori_loop(0, nch // 2, body, 0)
        _acc_epilogue(c, s, accum, out_hbm)

    return k(x, src2d, dstsc, zeros)


def _full(shape):
    return pl.BlockSpec(shape, lambda *_: tuple(0 for _ in shape))


def _time_body(t_ref, w1, b1, w2, b2, T0, bT0, T1, bT1, te0, te1):
    tcol = t_ref[...]
    j = lax.broadcasted_iota(jnp.int32, (1, TD), 1).astype(_f32)
    freqs = jnp.exp(-math.log(10000.0) * j / float(TD))
    a = tcol * freqs
    emb = jnp.concatenate([jnp.cos(a), jnp.sin(a)], axis=1)
    h = jnp.maximum(emb @ w1[...] + b1[...], 0.0)
    tm = h @ w2[...] + b2[...]
    te0[...] = tm @ T0[...] + bT0[...]
    te1[...] = tm @ T1[...] + bT1[...]


def _time_emb(tpad, tp, l0, l1):
    out = pl.pallas_call(
        _time_body,
        in_specs=[_full((16, 1)), _full((H, TD)), _full((1, TD)),
                  _full((TD, TD)), _full((1, TD)),
                  _full((TD, H)), _full((1, H)), _full((TD, H)), _full((1, H))],
        out_specs=[_full((16, H)), _full((16, H))],
        out_shape=[jax.ShapeDtypeStruct((16, H), _f32)] * 2,
    )
    return out(tpad, tp["W1"], tp["b1"][None, :], tp["W2"], tp["b2"][None, :],
               l0["T"], l0["bT"][None, :], l1["T"], l1["bT"][None, :])


def _embed_body(au_ref, W, b, o_ref):
    ji = lax.broadcasted_iota(jnp.int32, (1, H), 1)
    jf = jnp.floor(ji.astype(_f32) / 2.0)
    inv_dim_t = jnp.exp(-(math.log(10000.0) * 2.0 / float(H)) * jf)
    off = jnp.where(ji % 2 == 1, math.pi / 2.0, 0.0).astype(_f32)
    pos = au_ref[...] * inv_dim_t
    o_ref[...] = jnp.sin(pos + off) @ W[...] + b[...]


def _edge_embed(au, ep):
    out = pl.pallas_call(
        _embed_body,
        grid=(EP // EB,),
        in_specs=[pl.BlockSpec((EB, 1), lambda i: (i, 0)),
                  _full((H, H)), _full((1, H))],
        out_specs=pl.BlockSpec((EB, H), lambda i: (i, 0)),
        out_shape=jax.ShapeDtypeStruct((EP, H), _f32),
    )
    return out(au, ep["W"], ep["b"][None, :])


def _node_common(x, agg, W1, b1, W2, b2, eps, gw, gb, gms):
    h = (1.0 + eps[0, 0]) * x[...][0] + agg[...][0]
    h = jnp.maximum(h @ W1[...] + b1[...], 0.0)
    h = h @ W2[...] + b2[...]
    mean = jnp.mean(h, axis=0, keepdims=True)
    sub = h - mean * gms[...]
    var = jnp.mean(sub * sub, axis=0, keepdims=True)
    return jnp.maximum(gw[...] * sub * jax.lax.rsqrt(var + 1e-5) + gb[...],
                       0.0)


def _node0_body(x, agg, te, W1, b1, W2, b2, eps, gw, gb, gms,
                A, bA, B, bB, V, bV, U, bU,
                f_o, xav_o, xb_o, xu_o):
    f = _node_common(x, agg, W1, b1, W2, b2, eps, gw, gb, gms)
    f_o[...] = f[None]
    xa = f @ A[...] + bA[...] + te[...][0]
    xv = f @ V[...] + bV[...]
    xav_o[...] = jnp.concatenate([xa, xv], axis=1)[None]
    xb_o[...] = (f @ B[...] + bB[...])[None]
    xu_o[...] = (f @ U[...] + bU[...])[None]


def _node1_body(x, agg, te, W1, b1, W2, b2, eps, gw, gb, gms,
                A, bA, B, bB, xa_o, xb_o):
    f = _node_common(x, agg, W1, b1, W2, b2, eps, gw, gb, gms)
    xa_o[...] = (f @ A[...] + bA[...] + te[...][0])[None]
    xb_o[...] = (f @ B[...] + bB[...])[None]


def _node_dense(x, agg, te, lp, first):
    g = lp["gin"]
    n = lp["gn"]
    a = lp["agnn"]
    blk = pl.BlockSpec((1, HALF, H), lambda i: (i, 0, 0))
    blk2 = pl.BlockSpec((1, HALF, 2 * H), lambda i: (i, 0, 0))
    common_specs = [blk, blk, pl.BlockSpec((1, 1, H), lambda i: (i // 2, 0, 0)),
                    _full((H, H)), _full((1, H)), _full((H, H)), _full((1, H)),
                    _full((1, 1)), _full((1, H)), _full((1, H)), _full((1, H)),
                    _full((H, H)), _full((1, H)), _full((H, H)), _full((1, H))]
    common_args = (x.reshape(N // HALF, HALF, H),
                   agg.reshape(N // HALF, HALF, H), te.reshape(16, 1, H),
                   g["W1"], g["b1"][None, :], g["W2"], g["b2"][None, :],
                   g["eps"].reshape(1, 1),
                   n["weight"][None, :], n["bias"][None, :],
                   n["mean_scale"][None, :],
                   a["A"], a["bA"][None, :], a["B"], a["bB"][None, :])
    if first:
        out = pl.pallas_call(
            _node0_body,
            grid=(N // HALF,),
            in_specs=common_specs + [_full((H, H)), _full((1, H)),
                                     _full((H, H)), _full((1, H))],
            out_specs=[blk, blk2, blk, blk],
            out_shape=[jax.ShapeDtypeStruct((N // HALF, HALF, H), _f32),
                       jax.ShapeDtypeStruct((N // HALF, HALF, 2 * H), _f32),
                       jax.ShapeDtypeStruct((N // HALF, HALF, H), _f32),
                       jax.ShapeDtypeStruct((N // HALF, HALF, H), _f32)],
        )
        return out(*common_args, a["V"], a["bV"][None, :],
                   a["U"], a["bU"][None, :])
    out = pl.pallas_call(
        _node1_body,
        grid=(N // HALF,),
        in_specs=common_specs,
        out_specs=[blk, blk],
        out_shape=[jax.ShapeDtypeStruct((N // HALF, HALF, H), _f32)] * 2,
    )
    return out(*common_args)


def _edge0_body(e, gav, gb, C, bC, e1_o, msg_o):
    gav_v = gav[...]
    en = gav_v[:, :H] + gb[...] + e[...] @ C[...] + bC[...]
    gate = jax.nn.sigmoid(en)
    msg_o[...] = gate * gav_v[:, H:]
    e1_o[...] = e[...] + jnp.maximum(en, 0.0)


def _edge_dense0(e, gav, gb, a):
    blk = pl.BlockSpec((EB, H), lambda i: (i, 0))
    blk2 = pl.BlockSpec((EB, 2 * H), lambda i: (i, 0))
    out = pl.pallas_call(
        _edge0_body,
        grid=(EP // EB,),
        in_specs=[blk, blk2, blk, _full((H, H)), _full((1, H))],
        out_specs=[blk, blk],
        out_shape=[jax.ShapeDtypeStruct((EP, H), _f32)] * 2,
    )
    return out(e, gav, gb, a["C"], a["bC"][None, :])


def _edge1_body(e, ga, gb, C, bC, e1_o):
    en = ga[...] + gb[...] + e[...] @ C[...] + bC[...]
    e1_o[...] = e[...] + jnp.maximum(en, 0.0)


def _edge_dense1(e, ga, gb, a):
    blk = pl.BlockSpec((EB, H), lambda i: (i, 0))
    out = pl.pallas_call(
        _edge1_body,
        grid=(EP // EB,),
        in_specs=[blk, blk, blk, _full((H, H)), _full((1, H))],
        out_specs=blk,
        out_shape=jax.ShapeDtypeStruct((EP, H), _f32),
    )
    return out(e, ga, gb, a["C"], a["bC"][None, :])


def _xupd_body(f, xu, agg2, o):
    o[...] = f[...] + jnp.maximum(xu[...] + agg2[...], 0.0)


def _xupd(f, xu, agg2):
    blk = pl.BlockSpec((1, HALF, H), lambda i: (i, 0, 0))
    out = pl.pallas_call(
        _xupd_body,
        grid=(N // HALF,),
        in_specs=[blk, blk, blk],
        out_specs=blk,
        out_shape=jax.ShapeDtypeStruct((N // HALF, HALF, H), _f32),
    )
    return out(f, xu, agg2.reshape(N // HALF, HALF, H)).reshape(N, H)


def _head_body(ea, eb, W1, b1, W2, b2, W3, b3, o):
    def mlp(z):
        z = jnp.maximum(z @ W1[...] + b1[...], 0.0)
        z = jnp.maximum(z @ W2[...] + b2[...], 0.0)
        return z @ W3[...] + b3[...]

    o[...] = 0.5 * (mlp(ea[...]) + mlp(eb[...]))


def _head(e2, mp):
    out = pl.pallas_call(
        _head_body,
        grid=(EM // HB,),
        in_specs=[pl.BlockSpec((HB, H), lambda i: (i, 0)),
                  pl.BlockSpec((HB, H), lambda i: (i + EM // HB, 0)),
                  _full((H, 2 * H)), _full((1, 2 * H)),
                  _full((2 * H, H)), _full((1, H)),
                  _full((H, 1)), _full((1, 1))],
        out_specs=pl.BlockSpec((HB, 1), lambda i: (i, 0)),
        out_shape=jax.ShapeDtypeStruct((EM, 1), _f32),
    )
    return out(e2, e2, mp["W1"], mp["b1"][None, :], mp["W2"], mp["b2"][None, :],
               mp["W3"], mp["b3"].reshape(1, 1))


def kernel(x, edge_index, batch, x_indicator, edge_index_mapping,
           noise_mapping_attr, t, params):
    del batch, x_indicator
    padE = EP - EU
    zpad = jnp.zeros((padE,), jnp.int32)
    npad = jnp.full((padE,), N, jnp.int32)

    src_g = jnp.concatenate([edge_index[0], zpad]).reshape(IDXR, 128)
    dst_g = jnp.concatenate([edge_index[1], npad])
    src_u = jnp.concatenate(
        [edge_index_mapping[0], edge_index_mapping[1], zpad]).reshape(IDXR, 128)
    dst_u_raw = jnp.concatenate([edge_index_mapping[1], edge_index_mapping[0]])
    dst_u = jnp.concatenate([dst_u_raw, zpad]).reshape(IDXR, 128)
    dst_u_pad = jnp.concatenate([dst_u_raw, npad])

    def route(d):
        l0 = jnp.where(d < R, d, DUMMY)
        l1 = jnp.where((d >= R) & (d < N), d - R, DUMMY)
        return jnp.stack([l0, l1]).reshape(2, IDXR, 128)

    dstg_sc = route(dst_g)
    dstu_sc = route(dst_u_pad)

    au = jnp.concatenate([noise_mapping_attr, noise_mapping_attr,
                          jnp.zeros((padE,), _f32)]).reshape(EP, 1)
    tpad = jnp.pad(t, (0, 16 - G)).reshape(16, 1)
    zeros = jnp.zeros((512, H), _f32)

    lp0, lp1 = params["layers"]
    te0, te1 = _time_emb(tpad, params["time"], lp0["agnn"], lp1["agnn"])

    e = _edge_embed(au, params["edge_embed"])

    agg = _sc_gin_agg(x, src_g, dstg_sc, zeros)
    f0, xav0, xb0, xu0 = _node_dense(x, agg, te0, lp0, first=True)
    gav = _sc_gather(xav0.reshape(N, 2 * H), src_u, 2 * H)
    gb = _sc_gather(xb0.reshape(N, H), dst_u, H)
    e, msg = _edge_dense0(e, gav, gb, lp0["agnn"])
    agg2 = _sc_scatter_add(msg, dstu_sc, zeros)
    x1 = _xupd(f0, xu0, agg2)

    agg_b = _sc_gin_agg(x1, src_g, dstg_sc, zeros)
    xa1, xb1 = _node_dense(x1, agg_b, te1, lp1, first=False)
    ga1 = _sc_gather(xa1.reshape(N, H), src_u, H)
    gb1 = _sc_gather(xb1.reshape(N, H), dst_u, H)
    e = _edge_dense1(e, ga1, gb1, lp1["agnn"])

    return _head(e, params["map"])
